# baseline (speedup 1.0000x reference)
import jax
import jax.numpy as jnp
from jax import lax
from jax.experimental import pallas as pl
from jax.experimental.pallas import tpu as pltpu

N = 100000
D = 128
G = 1024
B = 10000
W = 128
NBLK = N // B


def _step(i, first_ref, nwin_ref, h_ref, ids_ref,
          w1a_ref, b1a_ref, w1b_ref, b1b_ref,
          acc_ref, write_ref, read_ref):
    x = jnp.dot(h_ref[...].astype(jnp.bfloat16),
                w1a_ref[...].astype(jnp.bfloat16),
                preferred_element_type=jnp.float32)
    x = jnp.maximum(x + b1a_ref[...], 0.0).astype(jnp.bfloat16)
    act = jnp.dot(x, w1b_ref[...].astype(jnp.bfloat16),
                  preferred_element_type=jnp.float32)
    write_ref[...] = jnp.maximum(act + b1b_ref[...], 0.0).astype(jnp.bfloat16)

    ids_row = ids_ref[0]
    first = first_ref[i]
    nwin = nwin_ref[i]

    row_iota = lax.broadcasted_iota(jnp.int32, (W, B), 0)
    act_prev = read_ref[...]

    def window(k):
        base = first + k * W
        rel = ids_row - base
        oh_t = (rel == row_iota).astype(jnp.bfloat16)
        partial = jnp.dot(oh_t, act_prev,
                          preferred_element_type=jnp.float32)
        acc_ref[pl.ds(base, W), :] += partial

    window(0)

    def body(k, carry):
        window(k)
        return carry

    lax.fori_loop(1, nwin, body, 0)


def _fused_kernel(first_ref, nwin_ref, h_ref, ids_ref,
                  w1a_ref, b1a_ref, w1b_ref, b1b_ref,
                  w2a_ref, b2a_ref, w2b_ref, b2b_ref,
                  out_ref, acc_ref, act0_ref, act1_ref):
    i = pl.program_id(0)

    @pl.when(i == 0)
    def _init():
        acc_ref[...] = jnp.zeros_like(acc_ref)

    args = (i, first_ref, nwin_ref, h_ref, ids_ref,
            w1a_ref, b1a_ref, w1b_ref, b1b_ref, acc_ref)

    @pl.when(i % 2 == 0)
    def _even():
        _step(*args, act0_ref, act1_ref)

    @pl.when(i % 2 == 1)
    def _odd():
        _step(*args, act1_ref, act0_ref)

    @pl.when(i == NBLK)
    def _finish():
        pooled = acc_ref[0:G, :]
        y = jnp.dot(pooled, w2a_ref[...], preferred_element_type=jnp.float32)
        y = jnp.maximum(y + b2a_ref[...], 0.0)
        z = jnp.dot(y, w2b_ref[...], preferred_element_type=jnp.float32)
        out_ref[...] = jnp.maximum(z + b2b_ref[...], 0.0)


@jax.jit
def kernel(h, graph_ids, W1a, b1a, W1b, b1b, W2a, b2a, W2b, b2b):
    ids32 = graph_ids.astype(jnp.int32)
    ids3 = ids32.reshape(NBLK, 1, B)
    firsts = ids32[::B]
    lasts = ids32[B - 1::B]
    nwin = (lasts - firsts) // W + 1
    sfirst = jnp.concatenate([jnp.full((1,), G, jnp.int32), firsts])
    snwin = jnp.concatenate([jnp.ones((1,), jnp.int32), nwin])

    full = lambda shape: pl.BlockSpec(shape, lambda i, *_: (0,) * len(shape))
    row = lambda: pl.BlockSpec((1, D), lambda i, *_: (0, 0))

    last_blk = NBLK - 1
    grid_spec = pltpu.PrefetchScalarGridSpec(
        num_scalar_prefetch=2,
        grid=(NBLK + 1,),
        in_specs=[
            pl.BlockSpec((B, D),
                         lambda i, *_: (jnp.minimum(i, last_blk), 0)),
            pl.BlockSpec((1, 1, B),
                         lambda i, *_: (jnp.maximum(i - 1, 0), 0, 0)),
            full((D, D)), row(), full((D, D)), row(),
            full((D, D)), row(), full((D, D)), row(),
        ],
        out_specs=pl.BlockSpec((G, D), lambda i, *_: (0, 0)),
        scratch_shapes=[
            pltpu.VMEM((G + W, D), jnp.float32),
            pltpu.VMEM((B, D), jnp.bfloat16),
            pltpu.VMEM((B, D), jnp.bfloat16),
        ],
    )

    return pl.pallas_call(
        _fused_kernel,
        grid_spec=grid_spec,
        out_shape=jax.ShapeDtypeStruct((G, D), jnp.float32),
    )(sfirst, snwin, h, ids3,
      W1a, b1a.reshape(1, D), W1b, b1b.reshape(1, D),
      W2a, b2a.reshape(1, D), W2b, b2b.reshape(1, D))


# speedup vs baseline: 1.4288x; 1.4288x over previous
import jax
import jax.numpy as jnp
from jax import lax
from jax.experimental import pallas as pl
from jax.experimental.pallas import tpu as pltpu

N = 100000
D = 128
G = 1024
B = 20000
NSUB = 5
BS = B // NSUB
W = 64
NSTEP = N // B


def _scatter_window(acc_ref, ids_row, act_bf, base):
    row_iota = lax.broadcasted_iota(jnp.int32, (W, BS), 0)
    oh_t = ((ids_row - base) == row_iota).astype(jnp.bfloat16)
    partial = jnp.dot(oh_t, act_bf, preferred_element_type=jnp.float32)
    acc_ref[pl.ds(base, W), :] += partial


def _fallback(acc_ref, first_ref, nwin_ref, ids_row, act_bf, g1):
    first = first_ref[g1]
    nwin = nwin_ref[g1]

    def body(k, carry):
        _scatter_window(acc_ref, ids_row, act_bf, first + k * W)
        return carry

    lax.fori_loop(1, nwin, body, 0)


def _fused_kernel(first_ref, nwin_ref, h_ref, ids_ref,
                  w1a_ref, b1a_ref, w1b_ref, b1b_ref,
                  w2a_ref, b2a_ref, w2b_ref, b2b_ref,
                  out_ref, acc_ref, acts_ref):
    i = pl.program_id(0)

    @pl.when(i == 0)
    def _init():
        acc_ref[...] = jnp.zeros_like(acc_ref)

    @pl.when(i < NSTEP)
    def _main():
        w1a = w1a_ref[...].astype(jnp.bfloat16)
        w1b = w1b_ref[...].astype(jnp.bfloat16)
        for s in range(NSUB):
            hb = h_ref[s * BS:(s + 1) * BS, :].astype(jnp.bfloat16)
            x = jnp.dot(hb, w1a, preferred_element_type=jnp.float32)
            x = jnp.maximum(x + b1a_ref[...], 0.0).astype(jnp.bfloat16)
            act = jnp.dot(x, w1b, preferred_element_type=jnp.float32)
            acts_ref[s] = jnp.maximum(
                act + b1b_ref[...], 0.0).astype(jnp.bfloat16)

            ids_row = ids_ref[0, :, s * BS:(s + 1) * BS]
            _scatter_window(acc_ref, ids_row,
                            acts_ref[(s - 1) % NSUB],
                            first_ref[i * NSUB + s])

        for s in range(NSUB):
            ids_row = ids_ref[0, :, s * BS:(s + 1) * BS]
            _fallback(acc_ref, first_ref, nwin_ref, ids_row,
                      acts_ref[(s - 1) % NSUB], i * NSUB + s)

    @pl.when(i == NSTEP)
    def _finish():
        ids_row = ids_ref[0, :, 0:BS]
        _scatter_window(acc_ref, ids_row, acts_ref[NSUB - 1],
                        first_ref[NSTEP * NSUB])
        _fallback(acc_ref, first_ref, nwin_ref, ids_row,
                  acts_ref[NSUB - 1], NSTEP * NSUB)

        pooled = acc_ref[0:G, :]
        y = jnp.dot(pooled, w2a_ref[...], preferred_element_type=jnp.float32)
        y = jnp.maximum(y + b2a_ref[...], 0.0)
        z = jnp.dot(y, w2b_ref[...], preferred_element_type=jnp.float32)
        out_ref[...] = jnp.maximum(z + b2b_ref[...], 0.0)


@jax.jit
def kernel(h, graph_ids, W1a, b1a, W1b, b1b, W2a, b2a, W2b, b2b):
    ids32 = graph_ids.astype(jnp.int32)
    padded = jnp.zeros(((NSTEP + 1) * B,), jnp.int32)
    padded = lax.dynamic_update_slice(padded, ids32, (BS,))
    ids3 = padded.reshape(NSTEP + 1, 1, B)

    firsts = ids32[::BS]
    lasts = ids32[BS - 1::BS]
    nwin = (lasts - firsts) // W + 1
    sfirst = jnp.concatenate([jnp.full((1,), G, jnp.int32), firsts])
    snwin = jnp.concatenate([jnp.ones((1,), jnp.int32), nwin])

    full = lambda shape: pl.BlockSpec(shape, lambda i, *_: (0,) * len(shape))
    row = lambda: pl.BlockSpec((1, D), lambda i, *_: (0, 0))

    last_blk = NSTEP - 1
    grid_spec = pltpu.PrefetchScalarGridSpec(
        num_scalar_prefetch=2,
        grid=(NSTEP + 1,),
        in_specs=[
            pl.BlockSpec((B, D),
                         lambda i, *_: (jnp.minimum(i, last_blk), 0)),
            pl.BlockSpec((1, 1, B), lambda i, *_: (i, 0, 0)),
            full((D, D)), row(), full((D, D)), row(),
            full((D, D)), row(), full((D, D)), row(),
        ],
        out_specs=pl.BlockSpec((G, D), lambda i, *_: (0, 0)),
        scratch_shapes=[
            pltpu.VMEM((G + W, D), jnp.float32),
            pltpu.VMEM((NSUB, BS, D), jnp.bfloat16),
        ],
    )

    return pl.pallas_call(
        _fused_kernel,
        grid_spec=grid_spec,
        out_shape=jax.ShapeDtypeStruct((G, D), jnp.float32),
    )(sfirst, snwin, h, ids3,
      W1a, b1a.reshape(1, D), W1b, b1b.reshape(1, D),
      W2a, b2a.reshape(1, D), W2b, b2b.reshape(1, D))


# speedup vs baseline: 1.4423x; 1.0095x over previous
import jax
import jax.numpy as jnp
from jax import lax
from jax.experimental import pallas as pl
from jax.experimental.pallas import tpu as pltpu

N = 100000
D = 128
G = 1024
B = 20000
NSUB = 5
BS = B // NSUB
W = 64
NSTEP = N // B


def _scatter_window(acc_ref, ids_row, act_bf, base):
    row_iota = lax.broadcasted_iota(jnp.int32, (W, BS), 0)
    oh_t = ((ids_row - base) == row_iota).astype(jnp.bfloat16)
    partial = jnp.dot(oh_t, act_bf, preferred_element_type=jnp.float32)
    acc_ref[pl.ds(base, W), :] += partial


def _fallback(acc_ref, first_ref, nwin_ref, ids_row, act_bf, g1):
    first = first_ref[g1]
    nwin = nwin_ref[g1]

    def body(k, carry):
        _scatter_window(acc_ref, ids_row, act_bf, first + k * W)
        return carry

    lax.fori_loop(1, nwin, body, 0)


def _fused_kernel(first_ref, nwin_ref, h_ref, ids_ref,
                  w1a_ref, b1a_ref, w1b_ref, b1b_ref,
                  w2a_ref, b2a_ref, w2b_ref, b2b_ref,
                  out_ref, acc_ref, acts_ref):
    i = pl.program_id(0)

    @pl.when(i == 0)
    def _init():
        acc_ref[...] = jnp.zeros_like(acc_ref)

    @pl.when(i < NSTEP)
    def _main():
        w1a = w1a_ref[...].astype(jnp.bfloat16)
        w1b = w1b_ref[...].astype(jnp.bfloat16)
        for s in range(NSUB):
            hb = h_ref[s * BS:(s + 1) * BS, :]
            x = jnp.dot(hb, w1a, preferred_element_type=jnp.float32)
            x = jnp.maximum(x + b1a_ref[...], 0.0).astype(jnp.bfloat16)
            act = jnp.dot(x, w1b, preferred_element_type=jnp.float32)
            acts_ref[s] = jnp.maximum(
                act + b1b_ref[...], 0.0).astype(jnp.bfloat16)

            ids_row = ids_ref[0, :, s * BS:(s + 1) * BS]
            _scatter_window(acc_ref, ids_row,
                            acts_ref[(s - 1) % NSUB],
                            first_ref[i * NSUB + s])

        for s in range(NSUB):
            ids_row = ids_ref[0, :, s * BS:(s + 1) * BS]
            _fallback(acc_ref, first_ref, nwin_ref, ids_row,
                      acts_ref[(s - 1) % NSUB], i * NSUB + s)

    @pl.when(i == NSTEP)
    def _finish():
        ids_row = ids_ref[0, :, 0:BS]
        _scatter_window(acc_ref, ids_row, acts_ref[NSUB - 1],
                        first_ref[NSTEP * NSUB])
        _fallback(acc_ref, first_ref, nwin_ref, ids_row,
                  acts_ref[NSUB - 1], NSTEP * NSUB)

        pooled = acc_ref[0:G, :]
        y = jnp.dot(pooled, w2a_ref[...], preferred_element_type=jnp.float32)
        y = jnp.maximum(y + b2a_ref[...], 0.0)
        z = jnp.dot(y, w2b_ref[...], preferred_element_type=jnp.float32)
        out_ref[...] = jnp.maximum(z + b2b_ref[...], 0.0)


@jax.jit
def kernel(h, graph_ids, W1a, b1a, W1b, b1b, W2a, b2a, W2b, b2b):
    ids32 = graph_ids.astype(jnp.int32)
    padded = jnp.zeros(((NSTEP + 1) * B,), jnp.int32)
    padded = lax.dynamic_update_slice(padded, ids32, (BS,))
    ids3 = padded.reshape(NSTEP + 1, 1, B)

    firsts = ids32[::BS]
    lasts = ids32[BS - 1::BS]
    nwin = (lasts - firsts) // W + 1
    sfirst = jnp.concatenate([jnp.full((1,), G, jnp.int32), firsts])
    snwin = jnp.concatenate([jnp.ones((1,), jnp.int32), nwin])

    full = lambda shape: pl.BlockSpec(shape, lambda i, *_: (0,) * len(shape))
    row = lambda: pl.BlockSpec((1, D), lambda i, *_: (0, 0))

    last_blk = NSTEP - 1
    grid_spec = pltpu.PrefetchScalarGridSpec(
        num_scalar_prefetch=2,
        grid=(NSTEP + 1,),
        in_specs=[
            pl.BlockSpec((B, D),
                         lambda i, *_: (jnp.minimum(i, last_blk), 0)),
            pl.BlockSpec((1, 1, B), lambda i, *_: (i, 0, 0)),
            full((D, D)), row(), full((D, D)), row(),
            full((D, D)), row(), full((D, D)), row(),
        ],
        out_specs=pl.BlockSpec((G, D), lambda i, *_: (0, 0)),
        scratch_shapes=[
            pltpu.VMEM((G + W, D), jnp.float32),
            pltpu.VMEM((NSUB, BS, D), jnp.bfloat16),
        ],
    )

    return pl.pallas_call(
        _fused_kernel,
        grid_spec=grid_spec,
        out_shape=jax.ShapeDtypeStruct((G, D), jnp.float32),
    )(sfirst, snwin, h, ids3,
      W1a, b1a.reshape(1, D), W1b, b1b.reshape(1, D),
      W2a, b2a.reshape(1, D), W2b, b2b.reshape(1, D))
